# baseline (speedup 1.0000x reference)
import jax
import jax.numpy as jnp
from jax.experimental import pallas as pl
from jax.experimental.pallas import tpu as pltpu

_TILE = 512
_CACHED_ROWS = 2048


def _cheb_fused_kernel(
    ltop_ref, lrest_ref, x0t_ref, w_ref, b_ref, out_ref, x0_scr, x1_scr
):
    k = pl.program_id(0)
    s = pl.program_id(1)
    f = x0t_ref.shape[0]
    cached = (s % 2) == 0
    row = jnp.where(cached, (s // 2) * _TILE, _CACHED_ROWS + (s // 2) * _TILE)

    @pl.when(jnp.logical_and(k == 0, s == 0))
    def _transpose_x0():
        x0_scr[...] = jnp.transpose(x0t_ref[...], (1, 0))

    def first_pass(l_tile):
        x1_scr[pl.ds(row, _TILE), :] = jnp.dot(
            l_tile, x0_scr[...], preferred_element_type=jnp.float32
        )

    def second_pass(l_tile):
        y = jnp.dot(l_tile, x1_scr[...], preferred_element_type=jnp.float32)
        x0_r = x0_scr[pl.ds(row, _TILE), :]
        x1_r = x1_scr[pl.ds(row, _TILE), :]
        x2_r = 2.0 * y - x0_r
        acc = jnp.dot(x0_r, w_ref[0:f, :], preferred_element_type=jnp.float32)
        acc += jnp.dot(x1_r, w_ref[f : 2 * f, :], preferred_element_type=jnp.float32)
        acc += jnp.dot(x2_r, w_ref[2 * f : 3 * f, :], preferred_element_type=jnp.float32)
        out_ref[...] = jnp.transpose(acc, (1, 0)) + b_ref[...]

    @pl.when(jnp.logical_and(k == 0, cached))
    def _():
        first_pass(ltop_ref[pl.ds((s // 2) * _TILE, _TILE), :])

    @pl.when(jnp.logical_and(k == 0, jnp.logical_not(cached)))
    def _():
        first_pass(lrest_ref[...])

    @pl.when(jnp.logical_and(k == 1, cached))
    def _():
        second_pass(ltop_ref[pl.ds((s // 2) * _TILE, _TILE), :])

    @pl.when(jnp.logical_and(k == 1, jnp.logical_not(cached)))
    def _():
        second_pass(lrest_ref[...])


def kernel(laplacian, inputs, weight, bias, precompute=0, einsum=0):
    B, Fin, V, X, Y, Z = inputs.shape
    K, _, Fout = weight.shape
    F = Fin * B * X * Y * Z

    x0t = inputs.reshape(F, V)
    w3 = weight.reshape(K * Fin, Fout)
    b2d = bias.reshape(Fout, 1)

    n_stream_tiles = (V - _CACHED_ROWS) // _TILE
    steps = 2 * n_stream_tiles

    def lrest_index(k, s):
        return ((_CACHED_ROWS // _TILE) * 2 + s) // 2, 0

    def out_index(k, s):
        tile_idx = jnp.where(
            (s % 2) == 0, s // 2, _CACHED_ROWS // _TILE + s // 2
        )
        return 0, jnp.where(k == 1, tile_idx, 0)

    out_t = pl.pallas_call(
        _cheb_fused_kernel,
        grid=(2, steps),
        in_specs=[
            pl.BlockSpec((_CACHED_ROWS, V), lambda k, s: (0, 0)),
            pl.BlockSpec((_TILE, V), lrest_index),
            pl.BlockSpec((F, V), lambda k, s: (0, 0)),
            pl.BlockSpec((K * F, Fout), lambda k, s: (0, 0)),
            pl.BlockSpec((Fout, 1), lambda k, s: (0, 0)),
        ],
        out_specs=pl.BlockSpec((Fout, _TILE), out_index),
        out_shape=jax.ShapeDtypeStruct((Fout, V), jnp.float32),
        scratch_shapes=[
            pltpu.VMEM((V, F), jnp.float32),
            pltpu.VMEM((V, F), jnp.float32),
        ],
    )(laplacian, laplacian, x0t, w3, b2d)

    return out_t.reshape(B, Fout, V, X, Y, Z)


# speedup vs baseline: 1.1770x; 1.1770x over previous
import jax
import jax.numpy as jnp
from jax.experimental import pallas as pl
from jax.experimental.pallas import tpu as pltpu

_C = 2048
_CTILE = 512
_CH = 256
_NBUF = 4


def _cheb_kernel(
    x0t_ref, w_ref, b_ref, l_hbm, out_ref, ltop_scr, x0_scr, x1_scr, bufs, sems, ltop_sem
):
    f = x0t_ref.shape[0]
    v = x0t_ref.shape[1]
    nstream = (v - _C) // _CH
    total = 2 * nstream
    ncached = _C // _CTILE

    def chunk_copy(i):
        row = _C + (i % nstream) * _CH
        slot = i % _NBUF
        return pltpu.make_async_copy(
            l_hbm.at[pl.ds(row, _CH), :],
            bufs.at[slot],
            sems.at[slot],
        )

    ltop_copy = pltpu.make_async_copy(l_hbm.at[pl.ds(0, _C), :], ltop_scr, ltop_sem)
    ltop_copy.start()
    for i in range(_NBUF - 1):
        chunk_copy(i).start()

    x0_scr[...] = jnp.transpose(x0t_ref[...], (1, 0))

    def second_pass_tile(l_tile, row, width):
        y = jnp.dot(l_tile, x1_scr[...], preferred_element_type=jnp.float32)
        x0_r = x0_scr[pl.ds(row, width), :]
        x1_r = x1_scr[pl.ds(row, width), :]
        x2_r = 2.0 * y - x0_r
        acc = jnp.dot(x0_r, w_ref[0:f, :], preferred_element_type=jnp.float32)
        acc += jnp.dot(x1_r, w_ref[f : 2 * f, :], preferred_element_type=jnp.float32)
        acc += jnp.dot(x2_r, w_ref[2 * f : 3 * f, :], preferred_element_type=jnp.float32)
        out_ref[:, pl.ds(row, width)] = jnp.transpose(acc, (1, 0)) + b_ref[...]

    ltop_copy.wait()
    for t in range(ncached):
        x1_scr[t * _CTILE : (t + 1) * _CTILE, :] = jnp.dot(
            ltop_scr[t * _CTILE : (t + 1) * _CTILE, :],
            x0_scr[...],
            preferred_element_type=jnp.float32,
        )

    def body(i, carry):
        slot = i % _NBUF
        chunk_copy(i).wait()

        @pl.when(i + _NBUF - 1 < total)
        def _issue_next():
            chunk_copy(i + _NBUF - 1).start()

        row = _C + (i % nstream) * _CH

        @pl.when(i < nstream)
        def _first_pass_chunk():
            x1_scr[pl.ds(row, _CH), :] = jnp.dot(
                bufs[slot], x0_scr[...], preferred_element_type=jnp.float32
            )

        @pl.when(i >= nstream)
        def _second_pass_chunk():
            second_pass_tile(bufs[slot], row, _CH)

        for t in range(ncached):
            @pl.when(i == nstream + t * (nstream // ncached))
            def _second_pass_cached():
                second_pass_tile(
                    ltop_scr[t * _CTILE : (t + 1) * _CTILE, :], t * _CTILE, _CTILE
                )

        return carry

    jax.lax.fori_loop(0, total, body, 0)


def kernel(laplacian, inputs, weight, bias, precompute=0, einsum=0):
    B, Fin, V, X, Y, Z = inputs.shape
    K, _, Fout = weight.shape
    F = Fin * B * X * Y * Z

    x0t = inputs.reshape(F, V)
    w3 = weight.reshape(K * Fin, Fout)
    b2d = bias.reshape(Fout, 1)

    out_t = pl.pallas_call(
        _cheb_kernel,
        in_specs=[
            pl.BlockSpec((F, V), lambda: (0, 0)),
            pl.BlockSpec((K * F, Fout), lambda: (0, 0)),
            pl.BlockSpec((Fout, 1), lambda: (0, 0)),
            pl.BlockSpec(memory_space=pl.ANY),
        ],
        out_specs=pl.BlockSpec((Fout, V), lambda: (0, 0)),
        out_shape=jax.ShapeDtypeStruct((Fout, V), jnp.float32),
        scratch_shapes=[
            pltpu.VMEM((_C, V), jnp.float32),
            pltpu.VMEM((V, F), jnp.float32),
            pltpu.VMEM((V, F), jnp.float32),
            pltpu.VMEM((_NBUF, _CH, V), jnp.float32),
            pltpu.SemaphoreType.DMA((_NBUF,)),
            pltpu.SemaphoreType.DMA,
        ],
    )(x0t, w3, b2d, laplacian)

    return out_t.reshape(B, Fout, V, X, Y, Z)
